# Initial kernel scaffold; baseline (speedup 1.0000x reference)
import jax
import jax.numpy as jnp
from jax.experimental import pallas as pl


def kernel(x, adj, adj_ad, Ws, As, W_out, a_out):
    raise NotImplementedError("write your pallas kernel here")


# speedup vs baseline: 1.3343x; 1.3343x over previous
import jax
import jax.numpy as jnp
from jax.experimental import pallas as pl

N = 4096
NFEAT = 512
NHID = 128
NCLASS = 64
NHEADS = 8
ALPHA = 0.2
BR = 256
NBLK = N // BR


def _proj_head(x_ref, w_ref, a_ref, wh_ref, f_ref):
    wh = jnp.dot(x_ref[...], w_ref[0], preferred_element_type=jnp.float32)
    wh_ref[0] = wh
    f_ref[0, 0] = jnp.dot(wh, a_ref[0, 0], preferred_element_type=jnp.float32)
    f_ref[0, 1] = jnp.dot(wh, a_ref[0, 1], preferred_element_type=jnp.float32)


def _attn_head(adj_ref, adjad_ref, wh_ref, f_ref, out_ref):
    i = pl.program_id(0)
    h = pl.program_id(1)
    maskpos = (adj_ref[...] + adjad_ref[...]) > 0.0
    f1 = f_ref[h, 0, pl.ds(i * BR, BR)]
    f2 = f_ref[h, 1, :]
    e = f1[:, None] + f2[None, :]
    e = jnp.where(e > 0.0, e, ALPHA * e)
    e = jnp.where(maskpos, e, jnp.float32(-9e15))
    m = jnp.max(e, axis=1, keepdims=True)
    p = jnp.exp(e - m)
    s = jnp.sum(p, axis=1, keepdims=True)
    att = p / s
    hp = jnp.dot(att, wh_ref[0], preferred_element_type=jnp.float32)
    out_ref[...] = jnp.where(hp > 0.0, hp, jnp.exp(hp) - 1.0)


def _proj_out(h_ref, w_ref, a_ref, who_ref, fo_ref):
    who = jnp.dot(h_ref[...], w_ref[...], preferred_element_type=jnp.float32)
    who_ref[...] = who
    fo_ref[0] = jnp.dot(who, a_ref[0], preferred_element_type=jnp.float32)
    fo_ref[1] = jnp.dot(who, a_ref[1], preferred_element_type=jnp.float32)


def _attn_out(adj_ref, adjad_ref, who_ref, fo_ref, out_ref):
    i = pl.program_id(0)
    maskpos = (adj_ref[...] + adjad_ref[...]) > 0.0
    f1 = fo_ref[0, pl.ds(i * BR, BR)]
    f2 = fo_ref[1, :]
    e = f1[:, None] + f2[None, :]
    e = jnp.where(e > 0.0, e, ALPHA * e)
    e = jnp.where(maskpos, e, jnp.float32(-9e15))
    m = jnp.max(e, axis=1, keepdims=True)
    p = jnp.exp(e - m)
    s = jnp.sum(p, axis=1, keepdims=True)
    att = p / s
    hp = jnp.dot(att, who_ref[...], preferred_element_type=jnp.float32)
    o = jnp.where(hp > 0.0, hp, jnp.exp(hp) - 1.0)
    mm = jnp.max(o, axis=1, keepdims=True)
    ls = o - mm
    out_ref[...] = ls - jnp.log(jnp.sum(jnp.exp(ls), axis=1, keepdims=True))


def kernel(x, adj, adj_ad, Ws, As, W_out, a_out):
    As3 = As.reshape(NHEADS, 2, NHID)
    ao2 = a_out.reshape(2, NCLASS)

    wh, f = pl.pallas_call(
        _proj_head,
        grid=(NHEADS,),
        in_specs=[
            pl.BlockSpec((N, NFEAT), lambda h: (0, 0)),
            pl.BlockSpec((1, NFEAT, NHID), lambda h: (h, 0, 0)),
            pl.BlockSpec((1, 2, NHID), lambda h: (h, 0, 0)),
        ],
        out_specs=[
            pl.BlockSpec((1, N, NHID), lambda h: (h, 0, 0)),
            pl.BlockSpec((1, 2, N), lambda h: (h, 0, 0)),
        ],
        out_shape=[
            jax.ShapeDtypeStruct((NHEADS, N, NHID), jnp.float32),
            jax.ShapeDtypeStruct((NHEADS, 2, N), jnp.float32),
        ],
    )(x, Ws, As3)

    h = pl.pallas_call(
        _attn_head,
        grid=(NBLK, NHEADS),
        in_specs=[
            pl.BlockSpec((BR, N), lambda i, hd: (i, 0)),
            pl.BlockSpec((BR, N), lambda i, hd: (i, 0)),
            pl.BlockSpec((1, N, NHID), lambda i, hd: (hd, 0, 0)),
            pl.BlockSpec((NHEADS, 2, N), lambda i, hd: (0, 0, 0)),
        ],
        out_specs=pl.BlockSpec((BR, NHID), lambda i, hd: (i, hd)),
        out_shape=jax.ShapeDtypeStruct((N, NHEADS * NHID), jnp.float32),
    )(adj, adj_ad, wh, f)

    who, fo = pl.pallas_call(
        _proj_out,
        in_specs=[
            pl.BlockSpec((N, NHEADS * NHID), lambda: (0, 0)),
            pl.BlockSpec((NHEADS * NHID, NCLASS), lambda: (0, 0)),
            pl.BlockSpec((2, NCLASS), lambda: (0, 0)),
        ],
        out_specs=[
            pl.BlockSpec((N, NCLASS), lambda: (0, 0)),
            pl.BlockSpec((2, N), lambda: (0, 0)),
        ],
        out_shape=[
            jax.ShapeDtypeStruct((N, NCLASS), jnp.float32),
            jax.ShapeDtypeStruct((2, N), jnp.float32),
        ],
    )(h, W_out, ao2)

    out = pl.pallas_call(
        _attn_out,
        grid=(NBLK,),
        in_specs=[
            pl.BlockSpec((BR, N), lambda i: (i, 0)),
            pl.BlockSpec((BR, N), lambda i: (i, 0)),
            pl.BlockSpec((N, NCLASS), lambda i: (0, 0)),
            pl.BlockSpec((2, N), lambda i: (0, 0)),
        ],
        out_specs=pl.BlockSpec((BR, NCLASS), lambda i: (i, 0)),
        out_shape=jax.ShapeDtypeStruct((N, NCLASS), jnp.float32),
    )(adj, adj_ad, who, fo)

    return out


# speedup vs baseline: 1.3835x; 1.0369x over previous
import jax
import jax.numpy as jnp
from jax.experimental import pallas as pl

N = 4096
NFEAT = 512
NHID = 128
NCLASS = 64
NHEADS = 8
ALPHA = 0.2
BR = 256
NBLK = N // BR


def _proj_head(x_ref, w_ref, a_ref, whb_ref, f_ref):
    wh = jnp.dot(x_ref[...], w_ref[0], preferred_element_type=jnp.float32)
    whb_ref[0] = wh.astype(jnp.bfloat16)
    f_ref[0, 0] = jnp.dot(wh, a_ref[0, 0], preferred_element_type=jnp.float32)
    f_ref[0, 1] = jnp.dot(wh, a_ref[0, 1], preferred_element_type=jnp.float32)


def _attn_head(adj_ref, adjad_ref, wh_ref, f_ref, out_ref):
    i = pl.program_id(0)
    h = pl.program_id(1)
    maskpos = (adj_ref[...] + adjad_ref[...]) > 0.0
    f1 = f_ref[h, 0, pl.ds(i * BR, BR)]
    f2 = f_ref[h, 1, :]
    e = f1[:, None] + f2[None, :]
    e = jnp.where(e > 0.0, e, ALPHA * e)
    e = jnp.where(maskpos, e, jnp.float32(-9e15))
    m = jnp.max(e, axis=1, keepdims=True)
    p = jnp.exp(e - m)
    s = jnp.sum(p, axis=1, keepdims=True)
    hp = jnp.dot(p.astype(jnp.bfloat16), wh_ref[0],
                 preferred_element_type=jnp.float32) * (1.0 / s)
    out_ref[...] = jnp.where(hp > 0.0, hp, jnp.exp(hp) - 1.0)


def _proj_out(h_ref, w_ref, a_ref, who_ref, fo_ref):
    who = jnp.dot(h_ref[...], w_ref[...], preferred_element_type=jnp.float32)
    who_ref[...] = who.astype(jnp.bfloat16)
    fo_ref[0] = jnp.dot(who, a_ref[0], preferred_element_type=jnp.float32)
    fo_ref[1] = jnp.dot(who, a_ref[1], preferred_element_type=jnp.float32)


def _attn_out(adj_ref, adjad_ref, who_ref, fo_ref, out_ref):
    i = pl.program_id(0)
    maskpos = (adj_ref[...] + adjad_ref[...]) > 0.0
    f1 = fo_ref[0, pl.ds(i * BR, BR)]
    f2 = fo_ref[1, :]
    e = f1[:, None] + f2[None, :]
    e = jnp.where(e > 0.0, e, ALPHA * e)
    e = jnp.where(maskpos, e, jnp.float32(-9e15))
    m = jnp.max(e, axis=1, keepdims=True)
    p = jnp.exp(e - m)
    s = jnp.sum(p, axis=1, keepdims=True)
    hp = jnp.dot(p.astype(jnp.bfloat16), who_ref[...],
                 preferred_element_type=jnp.float32) * (1.0 / s)
    o = jnp.where(hp > 0.0, hp, jnp.exp(hp) - 1.0)
    mm = jnp.max(o, axis=1, keepdims=True)
    ls = o - mm
    out_ref[...] = ls - jnp.log(jnp.sum(jnp.exp(ls), axis=1, keepdims=True))


def kernel(x, adj, adj_ad, Ws, As, W_out, a_out):
    As3 = As.reshape(NHEADS, 2, NHID)
    ao2 = a_out.reshape(2, NCLASS)

    whb, f = pl.pallas_call(
        _proj_head,
        grid=(NHEADS,),
        in_specs=[
            pl.BlockSpec((N, NFEAT), lambda h: (0, 0)),
            pl.BlockSpec((1, NFEAT, NHID), lambda h: (h, 0, 0)),
            pl.BlockSpec((1, 2, NHID), lambda h: (h, 0, 0)),
        ],
        out_specs=[
            pl.BlockSpec((1, N, NHID), lambda h: (h, 0, 0)),
            pl.BlockSpec((1, 2, N), lambda h: (h, 0, 0)),
        ],
        out_shape=[
            jax.ShapeDtypeStruct((NHEADS, N, NHID), jnp.bfloat16),
            jax.ShapeDtypeStruct((NHEADS, 2, N), jnp.float32),
        ],
    )(x, Ws, As3)

    h = pl.pallas_call(
        _attn_head,
        grid=(NBLK, NHEADS),
        in_specs=[
            pl.BlockSpec((BR, N), lambda i, hd: (i, 0)),
            pl.BlockSpec((BR, N), lambda i, hd: (i, 0)),
            pl.BlockSpec((1, N, NHID), lambda i, hd: (hd, 0, 0)),
            pl.BlockSpec((NHEADS, 2, N), lambda i, hd: (0, 0, 0)),
        ],
        out_specs=pl.BlockSpec((BR, NHID), lambda i, hd: (i, hd)),
        out_shape=jax.ShapeDtypeStruct((N, NHEADS * NHID), jnp.float32),
    )(adj, adj_ad, whb, f)

    who, fo = pl.pallas_call(
        _proj_out,
        in_specs=[
            pl.BlockSpec((N, NHEADS * NHID), lambda: (0, 0)),
            pl.BlockSpec((NHEADS * NHID, NCLASS), lambda: (0, 0)),
            pl.BlockSpec((2, NCLASS), lambda: (0, 0)),
        ],
        out_specs=[
            pl.BlockSpec((N, NCLASS), lambda: (0, 0)),
            pl.BlockSpec((2, N), lambda: (0, 0)),
        ],
        out_shape=[
            jax.ShapeDtypeStruct((N, NCLASS), jnp.bfloat16),
            jax.ShapeDtypeStruct((2, N), jnp.float32),
        ],
    )(h, W_out, ao2)

    out = pl.pallas_call(
        _attn_out,
        grid=(NBLK,),
        in_specs=[
            pl.BlockSpec((BR, N), lambda i: (i, 0)),
            pl.BlockSpec((BR, N), lambda i: (i, 0)),
            pl.BlockSpec((N, NCLASS), lambda i: (0, 0)),
            pl.BlockSpec((2, N), lambda i: (0, 0)),
        ],
        out_specs=pl.BlockSpec((BR, NCLASS), lambda i: (i, 0)),
        out_shape=jax.ShapeDtypeStruct((N, NCLASS), jnp.float32),
    )(adj, adj_ad, who, fo)

    return out


# speedup vs baseline: 2.5983x; 1.8781x over previous
import jax
import jax.numpy as jnp
from jax.experimental import pallas as pl

N = 4096
NFEAT = 512
NHID = 128
NCLASS = 64
NHEADS = 8
ALPHA = 0.2
BR = 256
NBLK = N // BR
NEG = -200.0


def _proj_head(x_ref, w_ref, a_ref, whb_ref, f_ref):
    wh = jnp.dot(x_ref[...], w_ref[0], preferred_element_type=jnp.float32)
    ones = jnp.ones((N, 1), jnp.float32)
    pad = jnp.zeros((N, 7), jnp.float32)
    whb_ref[0] = jnp.concatenate([wh, ones, pad], axis=1).astype(jnp.bfloat16)
    f_ref[0, 0] = jnp.dot(wh, a_ref[0, 0], preferred_element_type=jnp.float32)
    f_ref[0, 1] = jnp.dot(wh, a_ref[0, 1], preferred_element_type=jnp.float32)


def _attn_heads(adj_ref, adjad_ref, whb_ref, f_ref, out_ref, bias_ref):
    i = pl.program_id(0)
    bias = jnp.where(adj_ref[...] + adjad_ref[...] > 0.0, 0.0, NEG)
    bias_ref[...] = bias.astype(jnp.bfloat16)
    for h in range(NHEADS):
        f1 = f_ref[h, 0, pl.ds(i * BR, BR)]
        f2 = f_ref[h, 1, :]
        z = f1 + jnp.max(f_ref[h, 1, :])
        m = jnp.maximum(z, ALPHA * z)
        g1 = (f1 - m)[:, None]
        g2 = (ALPHA * f1 - m)[:, None]
        t = jnp.maximum(g1 + f2[None, :], g2 + (ALPHA * f2)[None, :])
        p = jnp.exp(t + bias).astype(jnp.bfloat16)
        acc = jnp.dot(p, whb_ref[h], preferred_element_type=jnp.float32)
        hp = acc[:, :NHID] * (1.0 / acc[:, NHID:NHID + 1])
        out_ref[:, h * NHID:(h + 1) * NHID] = jnp.where(
            hp > 0.0, hp, jnp.exp(hp) - 1.0)


def _proj_out(h_ref, w_ref, a_ref, who_ref, fo_ref):
    who = jnp.dot(h_ref[...], w_ref[...], preferred_element_type=jnp.float32)
    ones = jnp.ones((N, 1), jnp.float32)
    pad = jnp.zeros((N, 7), jnp.float32)
    who_ref[...] = jnp.concatenate([who, ones, pad], axis=1).astype(jnp.bfloat16)
    fo_ref[0] = jnp.dot(who, a_ref[0], preferred_element_type=jnp.float32)
    fo_ref[1] = jnp.dot(who, a_ref[1], preferred_element_type=jnp.float32)


def _attn_out(bias_ref, who_ref, fo_ref, out_ref):
    i = pl.program_id(0)
    f1 = fo_ref[0, pl.ds(i * BR, BR)]
    f2 = fo_ref[1, :]
    z = f1 + jnp.max(fo_ref[1, :])
    m = jnp.maximum(z, ALPHA * z)
    g1 = (f1 - m)[:, None]
    g2 = (ALPHA * f1 - m)[:, None]
    t = jnp.maximum(g1 + f2[None, :], g2 + (ALPHA * f2)[None, :])
    p = jnp.exp(t + bias_ref[...].astype(jnp.float32)).astype(jnp.bfloat16)
    acc = jnp.dot(p, who_ref[...], preferred_element_type=jnp.float32)
    hp = acc[:, :NCLASS] * (1.0 / acc[:, NCLASS:NCLASS + 1])
    o = jnp.where(hp > 0.0, hp, jnp.exp(hp) - 1.0)
    mm = jnp.max(o, axis=1, keepdims=True)
    ls = o - mm
    out_ref[...] = ls - jnp.log(jnp.sum(jnp.exp(ls), axis=1, keepdims=True))


def kernel(x, adj, adj_ad, Ws, As, W_out, a_out):
    As3 = As.reshape(NHEADS, 2, NHID)
    ao2 = a_out.reshape(2, NCLASS)

    whb, f = pl.pallas_call(
        _proj_head,
        grid=(NHEADS,),
        in_specs=[
            pl.BlockSpec((N, NFEAT), lambda h: (0, 0)),
            pl.BlockSpec((1, NFEAT, NHID), lambda h: (h, 0, 0)),
            pl.BlockSpec((1, 2, NHID), lambda h: (h, 0, 0)),
        ],
        out_specs=[
            pl.BlockSpec((1, N, NHID + 8), lambda h: (h, 0, 0)),
            pl.BlockSpec((1, 2, N), lambda h: (h, 0, 0)),
        ],
        out_shape=[
            jax.ShapeDtypeStruct((NHEADS, N, NHID + 8), jnp.bfloat16),
            jax.ShapeDtypeStruct((NHEADS, 2, N), jnp.float32),
        ],
    )(x, Ws, As3)

    h, bias = pl.pallas_call(
        _attn_heads,
        grid=(NBLK,),
        in_specs=[
            pl.BlockSpec((BR, N), lambda i: (i, 0)),
            pl.BlockSpec((BR, N), lambda i: (i, 0)),
            pl.BlockSpec((NHEADS, N, NHID + 8), lambda i: (0, 0, 0)),
            pl.BlockSpec((NHEADS, 2, N), lambda i: (0, 0, 0)),
        ],
        out_specs=[
            pl.BlockSpec((BR, NHEADS * NHID), lambda i: (i, 0)),
            pl.BlockSpec((BR, N), lambda i: (i, 0)),
        ],
        out_shape=[
            jax.ShapeDtypeStruct((N, NHEADS * NHID), jnp.float32),
            jax.ShapeDtypeStruct((N, N), jnp.bfloat16),
        ],
    )(adj, adj_ad, whb, f)

    who, fo = pl.pallas_call(
        _proj_out,
        in_specs=[
            pl.BlockSpec((N, NHEADS * NHID), lambda: (0, 0)),
            pl.BlockSpec((NHEADS * NHID, NCLASS), lambda: (0, 0)),
            pl.BlockSpec((2, NCLASS), lambda: (0, 0)),
        ],
        out_specs=[
            pl.BlockSpec((N, NCLASS + 8), lambda: (0, 0)),
            pl.BlockSpec((2, N), lambda: (0, 0)),
        ],
        out_shape=[
            jax.ShapeDtypeStruct((N, NCLASS + 8), jnp.bfloat16),
            jax.ShapeDtypeStruct((2, N), jnp.float32),
        ],
    )(h, W_out, ao2)

    out = pl.pallas_call(
        _attn_out,
        grid=(NBLK,),
        in_specs=[
            pl.BlockSpec((BR, N), lambda i: (i, 0)),
            pl.BlockSpec((N, NCLASS + 8), lambda i: (0, 0)),
            pl.BlockSpec((2, N), lambda i: (0, 0)),
        ],
        out_specs=pl.BlockSpec((BR, NCLASS), lambda i: (i, 0)),
        out_shape=jax.ShapeDtypeStruct((N, NCLASS), jnp.float32),
    )(bias, who, fo)

    return out


# speedup vs baseline: 2.8844x; 1.1101x over previous
import jax
import jax.numpy as jnp
from jax.experimental import pallas as pl
from jax.experimental.pallas import tpu as pltpu

N = 4096
NFEAT = 512
NHID = 128
NCLASS = 64
NHEADS = 8
ALPHA = 0.2
BR = 256
NBLK = N // BR
LOG2E = 1.4426950408889634
NEG = -256.0


def _proj_heads(x_ref, w_ref, a_ref, whb_ref, f_ref):
    x16 = x_ref[...].astype(jnp.bfloat16)
    ones = jnp.ones((N, 1), jnp.float32)
    pad = jnp.zeros((N, 7), jnp.float32)
    for h in range(NHEADS):
        wh = jnp.dot(x16, w_ref[h].astype(jnp.bfloat16),
                     preferred_element_type=jnp.float32)
        whb_ref[h] = jnp.concatenate([wh, ones, pad], axis=1).astype(jnp.bfloat16)
        f_ref[h, 0] = jnp.dot(wh, a_ref[h, 0],
                              preferred_element_type=jnp.float32) * LOG2E
        f_ref[h, 1] = jnp.dot(wh, a_ref[h, 1],
                              preferred_element_type=jnp.float32) * LOG2E


def _attn_heads(adj_ref, adjad_ref, whb_ref, f_ref, out_ref, bias_ref):
    i = pl.program_id(0)
    bias = jnp.where(adj_ref[...] + adjad_ref[...] > 0.0, 0.0, NEG)
    bias_ref[...] = bias.astype(jnp.bfloat16)
    for h in range(NHEADS):
        f1 = f_ref[h, 0, pl.ds(i * BR, BR)]
        f2 = f_ref[h, 1, :]
        z = f1 + jnp.max(f_ref[h, 1, :])
        m = jnp.maximum(z, ALPHA * z)
        g1 = (f1 - m)[:, None]
        g2 = (ALPHA * f1 - m)[:, None]
        t = jnp.maximum(g1 + f2[None, :], g2 + (ALPHA * f2)[None, :])
        p = jnp.exp2(t + bias).astype(jnp.bfloat16)
        acc = jnp.dot(p, whb_ref[h], preferred_element_type=jnp.float32)
        hp = acc[:, :NHID] * (1.0 / acc[:, NHID:NHID + 1])
        out_ref[:, h * NHID:(h + 1) * NHID] = jnp.where(
            hp > 0.0, hp, jnp.exp(hp) - 1.0).astype(jnp.bfloat16)


def _attn_out(bias_ref, h_ref, w_ref, a_ref, out_ref, who_s, fo_s):
    i = pl.program_id(0)

    @pl.when(i == 0)
    def _proj():
        who = jnp.dot(h_ref[...], w_ref[...].astype(jnp.bfloat16),
                      preferred_element_type=jnp.float32)
        ones = jnp.ones((N, 1), jnp.float32)
        pad = jnp.zeros((N, 7), jnp.float32)
        who_s[...] = jnp.concatenate([who, ones, pad], axis=1).astype(jnp.bfloat16)
        fo_s[0] = jnp.dot(who, a_ref[0], preferred_element_type=jnp.float32) * LOG2E
        fo_s[1] = jnp.dot(who, a_ref[1], preferred_element_type=jnp.float32) * LOG2E

    f1 = fo_s[0, pl.ds(i * BR, BR)]
    f2 = fo_s[1, :]
    z = f1 + jnp.max(fo_s[1, :])
    m = jnp.maximum(z, ALPHA * z)
    g1 = (f1 - m)[:, None]
    g2 = (ALPHA * f1 - m)[:, None]
    t = jnp.maximum(g1 + f2[None, :], g2 + (ALPHA * f2)[None, :])
    p = jnp.exp2(t + bias_ref[...].astype(jnp.float32)).astype(jnp.bfloat16)
    acc = jnp.dot(p, who_s[...], preferred_element_type=jnp.float32)
    hp = acc[:, :NCLASS] * (1.0 / acc[:, NCLASS:NCLASS + 1])
    o = jnp.where(hp > 0.0, hp, jnp.exp(hp) - 1.0)
    mm = jnp.max(o, axis=1, keepdims=True)
    ls = o - mm
    out_ref[...] = ls - jnp.log(jnp.sum(jnp.exp(ls), axis=1, keepdims=True))


def kernel(x, adj, adj_ad, Ws, As, W_out, a_out):
    As3 = As.reshape(NHEADS, 2, NHID)
    ao2 = a_out.reshape(2, NCLASS)

    whb, f = pl.pallas_call(
        _proj_heads,
        in_specs=[
            pl.BlockSpec((N, NFEAT), lambda: (0, 0)),
            pl.BlockSpec((NHEADS, NFEAT, NHID), lambda: (0, 0, 0)),
            pl.BlockSpec((NHEADS, 2, NHID), lambda: (0, 0, 0)),
        ],
        out_specs=[
            pl.BlockSpec((NHEADS, N, NHID + 8), lambda: (0, 0, 0)),
            pl.BlockSpec((NHEADS, 2, N), lambda: (0, 0, 0)),
        ],
        out_shape=[
            jax.ShapeDtypeStruct((NHEADS, N, NHID + 8), jnp.bfloat16),
            jax.ShapeDtypeStruct((NHEADS, 2, N), jnp.float32),
        ],
    )(x, Ws, As3)

    h, bias = pl.pallas_call(
        _attn_heads,
        grid=(NBLK,),
        in_specs=[
            pl.BlockSpec((BR, N), lambda i: (i, 0)),
            pl.BlockSpec((BR, N), lambda i: (i, 0)),
            pl.BlockSpec((NHEADS, N, NHID + 8), lambda i: (0, 0, 0)),
            pl.BlockSpec((NHEADS, 2, N), lambda i: (0, 0, 0)),
        ],
        out_specs=[
            pl.BlockSpec((BR, NHEADS * NHID), lambda i: (i, 0)),
            pl.BlockSpec((BR, N), lambda i: (i, 0)),
        ],
        out_shape=[
            jax.ShapeDtypeStruct((N, NHEADS * NHID), jnp.bfloat16),
            jax.ShapeDtypeStruct((N, N), jnp.bfloat16),
        ],
    )(adj, adj_ad, whb, f)

    out = pl.pallas_call(
        _attn_out,
        grid=(NBLK,),
        in_specs=[
            pl.BlockSpec((BR, N), lambda i: (i, 0)),
            pl.BlockSpec((N, NHEADS * NHID), lambda i: (0, 0)),
            pl.BlockSpec((NHEADS * NHID, NCLASS), lambda i: (0, 0)),
            pl.BlockSpec((2, NCLASS), lambda i: (0, 0)),
        ],
        out_specs=pl.BlockSpec((BR, NCLASS), lambda i: (i, 0)),
        out_shape=jax.ShapeDtypeStruct((N, NCLASS), jnp.float32),
        scratch_shapes=[
            pltpu.VMEM((N, NCLASS + 8), jnp.bfloat16),
            pltpu.VMEM((2, N), jnp.float32),
        ],
    )(bias, h, W_out, ao2)

    return out


# speedup vs baseline: 3.2055x; 1.1113x over previous
import jax
import jax.numpy as jnp
from jax.experimental import pallas as pl
from jax.experimental.pallas import tpu as pltpu

N = 4096
NFEAT = 512
NHID = 128
NCLASS = 64
NHEADS = 8
ALPHA = 0.2
BR = 256
NBLK = N // BR
LOG2E = 1.4426950408889634
NEG = -256.0


def _proj_heads(x_ref, w_ref, a_ref, whb_ref, f_ref, u_s):
    x16 = x_ref[...].astype(jnp.bfloat16)
    ones = jnp.ones((N, 1), jnp.float32)
    pad = jnp.zeros((N, 7), jnp.float32)
    for h in range(NHEADS):
        wh = jnp.dot(x16, w_ref[h].astype(jnp.bfloat16),
                     preferred_element_type=jnp.float32)
        whb_ref[h] = jnp.concatenate([wh, ones, pad], axis=1).astype(jnp.bfloat16)
        u_s[:, 2 * h:2 * h + 2] = jnp.dot(w_ref[h], a_ref[h],
                                          preferred_element_type=jnp.float32)
    F = jnp.dot(x16, u_s[...].astype(jnp.bfloat16),
                preferred_element_type=jnp.float32) * LOG2E
    f_ref[...] = F.T


def _attn_heads(adj_ref, adjad_ref, whb_ref, f_ref, out_ref, bias_ref):
    i = pl.program_id(0)
    bias = jnp.where(adj_ref[...] + adjad_ref[...] > 0.0, 0.0, NEG)
    bias_ref[...] = bias.astype(jnp.bfloat16)
    for h in range(NHEADS):
        f1 = f_ref[2 * h, pl.ds(i * BR, BR)]
        f2 = f_ref[2 * h + 1, :]
        z = f1 + jnp.max(f_ref[2 * h + 1, :])
        m = jnp.maximum(z, ALPHA * z)
        g1 = (f1 - m)[:, None]
        g2 = (ALPHA * f1 - m)[:, None]
        t = jnp.maximum(g1 + f2[None, :], g2 + (ALPHA * f2)[None, :])
        p = jnp.exp2(t + bias).astype(jnp.bfloat16)
        acc = jnp.dot(p, whb_ref[h], preferred_element_type=jnp.float32)
        hp = acc[:, :NHID] * (1.0 / acc[:, NHID:NHID + 1])
        out_ref[:, h * NHID:(h + 1) * NHID] = jnp.where(
            hp > 0.0, hp, jnp.exp(hp) - 1.0).astype(jnp.bfloat16)


def _attn_out(bias_ref, h_ref, w_ref, a_ref, out_ref, who_s, fo_s):
    i = pl.program_id(0)

    @pl.when(i == 0)
    def _proj():
        who = jnp.dot(h_ref[...], w_ref[...].astype(jnp.bfloat16),
                      preferred_element_type=jnp.float32)
        ones = jnp.ones((N, 1), jnp.float32)
        pad = jnp.zeros((N, 7), jnp.float32)
        who_s[...] = jnp.concatenate([who, ones, pad], axis=1).astype(jnp.bfloat16)
        fo = jnp.dot(who, a_ref[...], preferred_element_type=jnp.float32) * LOG2E
        fo_s[...] = fo.T

    f1 = fo_s[0, pl.ds(i * BR, BR)]
    f2 = fo_s[1, :]
    z = f1 + jnp.max(fo_s[1, :])
    m = jnp.maximum(z, ALPHA * z)
    g1 = (f1 - m)[:, None]
    g2 = (ALPHA * f1 - m)[:, None]
    t = jnp.maximum(g1 + f2[None, :], g2 + (ALPHA * f2)[None, :])
    p = jnp.exp2(t + bias_ref[...].astype(jnp.float32)).astype(jnp.bfloat16)
    acc = jnp.dot(p, who_s[...], preferred_element_type=jnp.float32)
    hp = acc[:, :NCLASS] * (1.0 / acc[:, NCLASS:NCLASS + 1])
    o = jnp.where(hp > 0.0, hp, jnp.exp(hp) - 1.0)
    mm = jnp.max(o, axis=1, keepdims=True)
    ls = o - mm
    out_ref[...] = ls - jnp.log(jnp.sum(jnp.exp(ls), axis=1, keepdims=True))


def kernel(x, adj, adj_ad, Ws, As, W_out, a_out):
    As4 = As.reshape(NHEADS, 2, NHID).transpose(0, 2, 1)
    ao2 = a_out.reshape(2, NCLASS).T

    whb, f = pl.pallas_call(
        _proj_heads,
        in_specs=[
            pl.BlockSpec((N, NFEAT), lambda: (0, 0)),
            pl.BlockSpec((NHEADS, NFEAT, NHID), lambda: (0, 0, 0)),
            pl.BlockSpec((NHEADS, NHID, 2), lambda: (0, 0, 0)),
        ],
        out_specs=[
            pl.BlockSpec((NHEADS, N, NHID + 8), lambda: (0, 0, 0)),
            pl.BlockSpec((2 * NHEADS, N), lambda: (0, 0)),
        ],
        out_shape=[
            jax.ShapeDtypeStruct((NHEADS, N, NHID + 8), jnp.bfloat16),
            jax.ShapeDtypeStruct((2 * NHEADS, N), jnp.float32),
        ],
        scratch_shapes=[pltpu.VMEM((NFEAT, 2 * NHEADS), jnp.float32)],
    )(x, Ws, As4)

    h, bias = pl.pallas_call(
        _attn_heads,
        grid=(NBLK,),
        in_specs=[
            pl.BlockSpec((BR, N), lambda i: (i, 0)),
            pl.BlockSpec((BR, N), lambda i: (i, 0)),
            pl.BlockSpec((NHEADS, N, NHID + 8), lambda i: (0, 0, 0)),
            pl.BlockSpec((2 * NHEADS, N), lambda i: (0, 0)),
        ],
        out_specs=[
            pl.BlockSpec((BR, NHEADS * NHID), lambda i: (i, 0)),
            pl.BlockSpec((BR, N), lambda i: (i, 0)),
        ],
        out_shape=[
            jax.ShapeDtypeStruct((N, NHEADS * NHID), jnp.bfloat16),
            jax.ShapeDtypeStruct((N, N), jnp.bfloat16),
        ],
    )(adj, adj_ad, whb, f)

    out = pl.pallas_call(
        _attn_out,
        grid=(NBLK,),
        in_specs=[
            pl.BlockSpec((BR, N), lambda i: (i, 0)),
            pl.BlockSpec((N, NHEADS * NHID), lambda i: (0, 0)),
            pl.BlockSpec((NHEADS * NHID, NCLASS), lambda i: (0, 0)),
            pl.BlockSpec((NCLASS, 2), lambda i: (0, 0)),
        ],
        out_specs=pl.BlockSpec((BR, NCLASS), lambda i: (i, 0)),
        out_shape=jax.ShapeDtypeStruct((N, NCLASS), jnp.float32),
        scratch_shapes=[
            pltpu.VMEM((N, NCLASS + 8), jnp.bfloat16),
            pltpu.VMEM((2, N), jnp.float32),
        ],
    )(bias, h, W_out, ao2)

    return out


# speedup vs baseline: 3.3545x; 1.0465x over previous
import jax
import jax.numpy as jnp
from jax.experimental import pallas as pl
from jax.experimental.pallas import tpu as pltpu

N = 4096
NFEAT = 512
NHID = 128
NCLASS = 64
NHEADS = 8
ALPHA = 0.2
BR = 256
NBLK = N // BR
LOG2E = 1.4426950408889634
NEG = -256.0


def _proj_heads(x_ref, w_ref, a_ref, whb_ref, f_ref, u_s):
    x16 = x_ref[...].astype(jnp.bfloat16)
    ones = jnp.ones((N, 1), jnp.float32)
    pad = jnp.zeros((N, 7), jnp.float32)
    for h in range(NHEADS):
        wh = jnp.dot(x16, w_ref[h].astype(jnp.bfloat16),
                     preferred_element_type=jnp.float32)
        whb_ref[h] = jnp.concatenate([wh, ones, pad], axis=1).astype(jnp.bfloat16)
        u_s[:, 2 * h:2 * h + 2] = jnp.dot(w_ref[h], a_ref[h],
                                          preferred_element_type=jnp.float32)
    F = jnp.dot(x16, u_s[...].astype(jnp.bfloat16),
                preferred_element_type=jnp.float32) * LOG2E
    f_ref[...] = F.T


def _attn_heads(adj_ref, adjad_ref, whb_ref, f_ref, out_ref, bias_ref):
    i = pl.program_id(0)
    m16 = jnp.where(adj_ref[...] + adjad_ref[...] > 0.0, 1.0, 0.0).astype(jnp.bfloat16)
    bias_ref[...] = m16
    for h in range(NHEADS):
        f1 = f_ref[2 * h, pl.ds(i * BR, BR)]
        f2 = f_ref[2 * h + 1, :]
        z = f1 + jnp.max(f_ref[2 * h + 1, :])
        m = jnp.maximum(z, ALPHA * z)
        g1 = (f1 - m)[:, None]
        g2 = (ALPHA * f1 - m)[:, None]
        t = jnp.maximum(g1 + f2[None, :], g2 + (ALPHA * f2)[None, :])
        p = jnp.exp2(t).astype(jnp.bfloat16) * m16
        acc = jnp.dot(p, whb_ref[h], preferred_element_type=jnp.float32)
        hp = acc[:, :NHID] * (1.0 / acc[:, NHID:NHID + 1])
        out_ref[:, h * NHID:(h + 1) * NHID] = jnp.where(
            hp > 0.0, hp, jnp.exp(hp) - 1.0).astype(jnp.bfloat16)


def _attn_out(bias_ref, h_ref, w_ref, a_ref, out_ref, who_s, fo_s):
    i = pl.program_id(0)

    @pl.when(i == 0)
    def _proj():
        who = jnp.dot(h_ref[...], w_ref[...].astype(jnp.bfloat16),
                      preferred_element_type=jnp.float32)
        ones = jnp.ones((N, 1), jnp.float32)
        pad = jnp.zeros((N, 7), jnp.float32)
        who_s[...] = jnp.concatenate([who, ones, pad], axis=1).astype(jnp.bfloat16)
        fo = jnp.dot(who, a_ref[...], preferred_element_type=jnp.float32) * LOG2E
        fo_s[...] = fo.T

    f1 = fo_s[0, pl.ds(i * BR, BR)]
    f2 = fo_s[1, :]
    z = f1 + jnp.max(fo_s[1, :])
    m = jnp.maximum(z, ALPHA * z)
    g1 = (f1 - m)[:, None]
    g2 = (ALPHA * f1 - m)[:, None]
    t = jnp.maximum(g1 + f2[None, :], g2 + (ALPHA * f2)[None, :])
    p = jnp.exp2(t).astype(jnp.bfloat16) * bias_ref[...]
    acc = jnp.dot(p, who_s[...], preferred_element_type=jnp.float32)
    hp = acc[:, :NCLASS] * (1.0 / acc[:, NCLASS:NCLASS + 1])
    o = jnp.where(hp > 0.0, hp, jnp.exp(hp) - 1.0)
    mm = jnp.max(o, axis=1, keepdims=True)
    ls = o - mm
    out_ref[...] = ls - jnp.log(jnp.sum(jnp.exp(ls), axis=1, keepdims=True))


def kernel(x, adj, adj_ad, Ws, As, W_out, a_out):
    As4 = As.reshape(NHEADS, 2, NHID).transpose(0, 2, 1)
    ao2 = a_out.reshape(2, NCLASS).T

    whb, f = pl.pallas_call(
        _proj_heads,
        in_specs=[
            pl.BlockSpec((N, NFEAT), lambda: (0, 0)),
            pl.BlockSpec((NHEADS, NFEAT, NHID), lambda: (0, 0, 0)),
            pl.BlockSpec((NHEADS, NHID, 2), lambda: (0, 0, 0)),
        ],
        out_specs=[
            pl.BlockSpec((NHEADS, N, NHID + 8), lambda: (0, 0, 0)),
            pl.BlockSpec((2 * NHEADS, N), lambda: (0, 0)),
        ],
        out_shape=[
            jax.ShapeDtypeStruct((NHEADS, N, NHID + 8), jnp.bfloat16),
            jax.ShapeDtypeStruct((2 * NHEADS, N), jnp.float32),
        ],
        scratch_shapes=[pltpu.VMEM((NFEAT, 2 * NHEADS), jnp.float32)],
    )(x, Ws, As4)

    h, bias = pl.pallas_call(
        _attn_heads,
        grid=(NBLK,),
        in_specs=[
            pl.BlockSpec((BR, N), lambda i: (i, 0)),
            pl.BlockSpec((BR, N), lambda i: (i, 0)),
            pl.BlockSpec((NHEADS, N, NHID + 8), lambda i: (0, 0, 0)),
            pl.BlockSpec((2 * NHEADS, N), lambda i: (0, 0)),
        ],
        out_specs=[
            pl.BlockSpec((BR, NHEADS * NHID), lambda i: (i, 0)),
            pl.BlockSpec((BR, N), lambda i: (i, 0)),
        ],
        out_shape=[
            jax.ShapeDtypeStruct((N, NHEADS * NHID), jnp.bfloat16),
            jax.ShapeDtypeStruct((N, N), jnp.bfloat16),
        ],
    )(adj, adj_ad, whb, f)

    out = pl.pallas_call(
        _attn_out,
        grid=(NBLK,),
        in_specs=[
            pl.BlockSpec((BR, N), lambda i: (i, 0)),
            pl.BlockSpec((N, NHEADS * NHID), lambda i: (0, 0)),
            pl.BlockSpec((NHEADS * NHID, NCLASS), lambda i: (0, 0)),
            pl.BlockSpec((NCLASS, 2), lambda i: (0, 0)),
        ],
        out_specs=pl.BlockSpec((BR, NCLASS), lambda i: (i, 0)),
        out_shape=jax.ShapeDtypeStruct((N, NCLASS), jnp.float32),
        scratch_shapes=[
            pltpu.VMEM((N, NCLASS + 8), jnp.bfloat16),
            pltpu.VMEM((2, N), jnp.float32),
        ],
    )(bias, h, W_out, ao2)

    return out
